# baseline (speedup 1.0000x reference)
import functools

import jax
import jax.numpy as jnp
import numpy as np
from jax.experimental import pallas as pl
from jax.experimental.pallas import tpu as pltpu

P_RESAMPLE = 0.2
TN = 256
CAP = 2048

_ROT = ((13, 15, 26, 6), (17, 29, 16, 24))
_TINY = np.float32(np.finfo(np.float32).tiny)


def _gumbel_bits(i):
    k0 = jnp.uint32(0)
    k1 = jnp.uint32(42)
    ks = (k0, k1, k0 ^ k1 ^ jnp.uint32(0x1BD11BDA))
    x0 = jnp.full_like(i, ks[0])
    x1 = i + ks[1]
    for g in range(5):
        for r in _ROT[g % 2]:
            x0 = x0 + x1
            x1 = ((x1 << jnp.uint32(r)) | (x1 >> jnp.uint32(32 - r))) ^ x0
        x0 = x0 + ks[(g + 1) % 3]
        x1 = x1 + ks[(g + 2) % 3] + jnp.uint32(g + 1)
    return x0 ^ x1


def _gumbel(i):
    bits = _gumbel_bits(i)
    fb = (bits >> jnp.uint32(9)) | jnp.uint32(0x3F800000)
    f = jax.lax.bitcast_convert_type(fb, jnp.float32) - jnp.float32(1.0)
    u = jnp.maximum(_TINY, f * (jnp.float32(1.0) - _TINY) + _TINY)
    return -jnp.log(-jnp.log(u))


def _sample_kernel(n_total, counts_ref, toks_ref, idx_ref, sq_ref, embs_ref,
                   out_ref):
    tn = toks_ref.shape[-1]
    v = embs_ref.shape[1]
    ci = pl.program_id(0)
    ji = pl.program_id(1)

    @pl.when(ji * tn < counts_ref[ci])
    def _():
        _sample_tile(n_total, ci, toks_ref, idx_ref, sq_ref, embs_ref, out_ref)


def _sample_tile(n_total, ci, toks_ref, idx_ref, sq_ref, embs_ref, out_ref):
    tn = toks_ref.shape[-1]
    v = embs_ref.shape[1]
    tok = toks_ref[0, 0, :]
    n_idx = idx_ref[0, 0, :]
    embs_c = embs_ref[0]
    sq_c = sq_ref[0, 0, :]

    iota_v = jax.lax.broadcasted_iota(jnp.int32, (tn, v), 1)
    is_tok = iota_v == tok[:, None]
    onehot = is_tok.astype(jnp.float32)

    ge = jax.lax.dot_general(
        onehot, embs_c, (((1,), (0,)), ((), ())),
        precision=jax.lax.Precision.HIGHEST)
    sq_tok = jax.lax.dot_general(
        onehot, sq_c[:, None], (((1,), (0,)), ((), ())),
        precision=jax.lax.Precision.HIGHEST)

    inner = jax.lax.dot_general(
        ge, embs_c, (((1,), (1,)), ((), ())),
        precision=jax.lax.Precision.DEFAULT)

    d2 = (sq_tok + sq_c[None, :]) - 2.0 * inner
    dist = jnp.sqrt(jnp.maximum(d2, 0.0))
    logits = jnp.where(is_tok, -jnp.inf, -dist)

    row = ci * n_total + n_idx
    base = row.astype(jnp.uint32) * jnp.uint32(v)
    flat_i = base[:, None] + iota_v.astype(jnp.uint32)
    score = logits + _gumbel(flat_i)

    m = jnp.max(score, axis=1, keepdims=True)
    out_ref[0, 0, :] = jnp.min(jnp.where(score == m, iota_v, v), axis=1)


def kernel(toks, embs):
    b, t, c = toks.shape
    _, v, d = embs.shape
    n = b * t
    nb = CAP // TN

    toks_cn = toks.reshape(n, c).T
    sq = jnp.sum(embs * embs, axis=-1).reshape(c, 1, v)
    u = jax.random.uniform(jax.random.key(7), (b, t, c))
    mask_cn = (u < P_RESAMPLE).reshape(n, c).T

    idx = jnp.argsort(~mask_cn, axis=1, stable=True)[:, :CAP]
    valid = jnp.take_along_axis(mask_cn, idx, axis=1)
    toks_sel = jnp.take_along_axis(toks_cn, idx, axis=1)
    counts = jnp.sum(mask_cn, axis=1, dtype=jnp.int32)

    samples = pl.pallas_call(
        functools.partial(_sample_kernel, n),
        grid_spec=pltpu.PrefetchScalarGridSpec(
            num_scalar_prefetch=1,
            grid=(c, nb),
            in_specs=[
                pl.BlockSpec((1, 1, TN), lambda ci, i, cnt: (ci * nb + i, 0, 0)),
                pl.BlockSpec((1, 1, TN), lambda ci, i, cnt: (ci * nb + i, 0, 0)),
                pl.BlockSpec((1, 1, v), lambda ci, i, cnt: (ci, 0, 0)),
                pl.BlockSpec((1, v, d), lambda ci, i, cnt: (ci, 0, 0)),
            ],
            out_specs=pl.BlockSpec(
                (1, 1, TN), lambda ci, i, cnt: (ci * nb + i, 0, 0)),
        ),
        out_shape=jax.ShapeDtypeStruct((c * nb, 1, TN), jnp.int32),
    )(counts, toks_sel.reshape(c * nb, 1, TN), idx.reshape(c * nb, 1, TN),
      sq, embs)

    samples = samples.reshape(c, CAP)
    scatter_idx = jnp.where(valid, idx, n)
    new_cn = toks_cn.at[jnp.arange(c)[:, None], scatter_idx].set(
        samples, mode='drop', unique_indices=True)
    return new_cn.T.reshape(b, t, c)


# speedup vs baseline: 1.1467x; 1.1467x over previous
import functools

import jax
import jax.numpy as jnp
import numpy as np
from jax.experimental import pallas as pl
from jax.experimental.pallas import tpu as pltpu

P_RESAMPLE = 0.2
TN = 256
CAP = 2048

_ROT = ((13, 15, 26, 6), (17, 29, 16, 24))
_TINY = np.float32(np.finfo(np.float32).tiny)


def _gumbel_bits(i):
    k0 = jnp.uint32(0)
    k1 = jnp.uint32(42)
    ks = (k0, k1, k0 ^ k1 ^ jnp.uint32(0x1BD11BDA))
    x0 = jnp.full_like(i, ks[0])
    x1 = i + ks[1]
    for g in range(5):
        for r in _ROT[g % 2]:
            x0 = x0 + x1
            x1 = ((x1 << jnp.uint32(r)) | (x1 >> jnp.uint32(32 - r))) ^ x0
        x0 = x0 + ks[(g + 1) % 3]
        x1 = x1 + ks[(g + 2) % 3] + jnp.uint32(g + 1)
    return x0 ^ x1


def _gumbel(i):
    bits = _gumbel_bits(i)
    fb = (bits >> jnp.uint32(9)) | jnp.uint32(0x3F800000)
    f = jax.lax.bitcast_convert_type(fb, jnp.float32) - jnp.float32(1.0)
    u = jnp.maximum(_TINY, f * (jnp.float32(1.0) - _TINY) + _TINY)
    return -jnp.log(-jnp.log(u))


def _sample_kernel(n_total, counts_ref, toks_ref, idx_ref, sq_ref, embs_ref,
                   out_ref):
    tn = toks_ref.shape[-1]
    v = embs_ref.shape[1]
    ci = pl.program_id(0)
    ji = pl.program_id(1)

    @pl.when(ji * tn < counts_ref[ci])
    def _():
        _sample_tile(n_total, ci, toks_ref, idx_ref, sq_ref, embs_ref, out_ref)


def _sample_tile(n_total, ci, toks_ref, idx_ref, sq_ref, embs_ref, out_ref):
    tn = toks_ref.shape[-1]
    v = embs_ref.shape[1]
    tok = toks_ref[0, 0, :]
    n_idx = idx_ref[0, 0, :]
    embs_c = embs_ref[0]
    sq_c = sq_ref[0, 0, :]

    iota_v = jax.lax.broadcasted_iota(jnp.int32, (tn, v), 1)
    is_tok = iota_v == tok[:, None]
    onehot = is_tok.astype(jnp.float32)

    ge = jax.lax.dot_general(
        onehot, embs_c, (((1,), (0,)), ((), ())),
        precision=jax.lax.Precision.HIGHEST)
    sq_tok = jnp.sum(jnp.where(is_tok, sq_c[None, :], 0.0), axis=1,
                     keepdims=True)

    inner = jax.lax.dot_general(
        ge, embs_c, (((1,), (1,)), ((), ())),
        precision=jax.lax.Precision.DEFAULT)

    d2 = (sq_tok + sq_c[None, :]) - 2.0 * inner
    dist = jnp.sqrt(jnp.maximum(d2, 0.0))
    logits = jnp.where(is_tok, -jnp.inf, -dist)

    row = ci * n_total + n_idx
    base = row.astype(jnp.uint32) * jnp.uint32(v)
    flat_i = base[:, None] + iota_v.astype(jnp.uint32)
    score = logits + _gumbel(flat_i)

    m = jnp.max(score, axis=1, keepdims=True)
    out_ref[0, 0, :] = jnp.min(jnp.where(score == m, iota_v, v), axis=1)


def kernel(toks, embs):
    b, t, c = toks.shape
    _, v, d = embs.shape
    n = b * t
    nb = CAP // TN

    toks_cn = toks.reshape(n, c).T
    sq = jnp.sum(embs * embs, axis=-1).reshape(c, 1, v)
    u = jax.random.uniform(jax.random.key(7), (b, t, c))
    mask_cn = (u < P_RESAMPLE).reshape(n, c).T

    n_iota = jax.lax.broadcasted_iota(jnp.int32, (c, n), 1)
    key = jnp.where(mask_cn, n_iota, n_iota + n)
    skey = jax.lax.sort(key, dimension=1)[:, :CAP]
    idx = skey & (n - 1)
    valid = skey < n
    toks_sel = jnp.take_along_axis(toks_cn, idx, axis=1)
    counts = jnp.sum(mask_cn, axis=1, dtype=jnp.int32)

    samples = pl.pallas_call(
        functools.partial(_sample_kernel, n),
        grid_spec=pltpu.PrefetchScalarGridSpec(
            num_scalar_prefetch=1,
            grid=(c, nb),
            in_specs=[
                pl.BlockSpec((1, 1, TN), lambda ci, i, cnt: (ci * nb + i, 0, 0)),
                pl.BlockSpec((1, 1, TN), lambda ci, i, cnt: (ci * nb + i, 0, 0)),
                pl.BlockSpec((1, 1, v), lambda ci, i, cnt: (ci, 0, 0)),
                pl.BlockSpec((1, v, d), lambda ci, i, cnt: (ci, 0, 0)),
            ],
            out_specs=pl.BlockSpec(
                (1, 1, TN), lambda ci, i, cnt: (ci * nb + i, 0, 0)),
        ),
        out_shape=jax.ShapeDtypeStruct((c * nb, 1, TN), jnp.int32),
    )(counts, toks_sel.reshape(c * nb, 1, TN), idx.reshape(c * nb, 1, TN),
      sq, embs)

    samples = samples.reshape(c, CAP)
    scatter_idx = jnp.where(valid, idx, n)
    new_cn = toks_cn.at[jnp.arange(c)[:, None], scatter_idx].set(
        samples, mode='drop', unique_indices=True)
    return new_cn.T.reshape(b, t, c)


# speedup vs baseline: 1.1472x; 1.0005x over previous
import functools

import jax
import jax.numpy as jnp
import numpy as np
from jax.experimental import pallas as pl
from jax.experimental.pallas import tpu as pltpu

P_RESAMPLE = 0.2
TN = 256
CAP = 2048

_ROT = ((13, 15, 26, 6), (17, 29, 16, 24))
_TINY = np.float32(np.finfo(np.float32).tiny)


def _gumbel_bits(i):
    k0 = jnp.uint32(0)
    k1 = jnp.uint32(42)
    ks = (k0, k1, k0 ^ k1 ^ jnp.uint32(0x1BD11BDA))
    x1 = i + ks[1]
    x0 = x1
    x1 = ((x1 << jnp.uint32(13)) | (x1 >> jnp.uint32(32 - 13))) ^ x0
    for g in range(5):
        for r in _ROT[g % 2][1:] if g == 0 else _ROT[g % 2]:
            x0 = x0 + x1
            x1 = ((x1 << jnp.uint32(r)) | (x1 >> jnp.uint32(32 - r))) ^ x0
        x0 = x0 + ks[(g + 1) % 3]
        x1 = x1 + ks[(g + 2) % 3] + jnp.uint32(g + 1)
    return x0 ^ x1


def _gumbel(i):
    bits = _gumbel_bits(i)
    fb = (bits >> jnp.uint32(9)) | jnp.uint32(0x3F800000)
    f = jax.lax.bitcast_convert_type(fb, jnp.float32) - jnp.float32(1.0)
    u = jnp.maximum(_TINY, f * (jnp.float32(1.0) - _TINY) + _TINY)
    return -jnp.log(-jnp.log(u))


def _sample_kernel(n_total, counts_ref, toks_ref, idx_ref, sq_ref, embs_ref,
                   out_ref):
    tn = toks_ref.shape[-1]
    v = embs_ref.shape[1]
    ci = pl.program_id(0)
    ji = pl.program_id(1)

    @pl.when(ji * tn < counts_ref[ci])
    def _():
        _sample_tile(n_total, ci, toks_ref, idx_ref, sq_ref, embs_ref, out_ref)


def _sample_tile(n_total, ci, toks_ref, idx_ref, sq_ref, embs_ref, out_ref):
    tn = toks_ref.shape[-1]
    v = embs_ref.shape[1]
    tok = toks_ref[0, 0, :]
    n_idx = idx_ref[0, 0, :]
    embs_c = embs_ref[0]
    sq_c = sq_ref[0, 0, :]

    iota_v = jax.lax.broadcasted_iota(jnp.int32, (tn, v), 1)
    is_tok = iota_v == tok[:, None]
    onehot = is_tok.astype(jnp.float32)

    ge = jax.lax.dot_general(
        onehot, embs_c, (((1,), (0,)), ((), ())),
        precision=jax.lax.Precision.HIGHEST)
    sq_tok = jnp.sum(jnp.where(is_tok, sq_c[None, :], 0.0), axis=1,
                     keepdims=True)

    inner = jax.lax.dot_general(
        ge, embs_c, (((1,), (1,)), ((), ())),
        precision=jax.lax.Precision.DEFAULT)

    d2 = (sq_tok + sq_c[None, :]) - 2.0 * inner
    dist = jnp.sqrt(jnp.maximum(d2, 0.0))
    logits = jnp.where(is_tok, -jnp.inf, -dist)

    row = ci * n_total + n_idx
    base = row.astype(jnp.uint32) * jnp.uint32(v)
    flat_i = base[:, None] + iota_v.astype(jnp.uint32)
    score = logits + _gumbel(flat_i)

    m = jnp.max(score, axis=1, keepdims=True)
    out_ref[0, 0, :] = jnp.min(jnp.where(score == m, iota_v, v), axis=1)


def kernel(toks, embs):
    b, t, c = toks.shape
    _, v, d = embs.shape
    n = b * t
    nb = CAP // TN

    toks_cn = toks.reshape(n, c).T
    sq = jnp.sum(embs * embs, axis=-1).reshape(c, 1, v)
    u = jax.random.uniform(jax.random.key(7), (b, t, c))
    mask_cn = (u < P_RESAMPLE).reshape(n, c).T

    n_iota = jax.lax.broadcasted_iota(jnp.int32, (c, n), 1)
    key = jnp.where(mask_cn, n_iota, n_iota + n)
    skey = jax.lax.sort(key, dimension=1)[:, :CAP]
    idx = skey & (n - 1)
    valid = skey < n
    toks_sel = jnp.take_along_axis(toks_cn, idx, axis=1)
    counts = jnp.sum(mask_cn, axis=1, dtype=jnp.int32)

    samples = pl.pallas_call(
        functools.partial(_sample_kernel, n),
        grid_spec=pltpu.PrefetchScalarGridSpec(
            num_scalar_prefetch=1,
            grid=(c, nb),
            in_specs=[
                pl.BlockSpec((1, 1, TN), lambda ci, i, cnt: (ci * nb + i, 0, 0)),
                pl.BlockSpec((1, 1, TN), lambda ci, i, cnt: (ci * nb + i, 0, 0)),
                pl.BlockSpec((1, 1, v), lambda ci, i, cnt: (ci, 0, 0)),
                pl.BlockSpec((1, v, d), lambda ci, i, cnt: (ci, 0, 0)),
            ],
            out_specs=pl.BlockSpec(
                (1, 1, TN), lambda ci, i, cnt: (ci * nb + i, 0, 0)),
        ),
        out_shape=jax.ShapeDtypeStruct((c * nb, 1, TN), jnp.int32),
    )(counts, toks_sel.reshape(c * nb, 1, TN), idx.reshape(c * nb, 1, TN),
      sq, embs)

    samples = samples.reshape(c, CAP)
    scatter_idx = jnp.where(valid, idx, n)
    new_cn = toks_cn.at[jnp.arange(c)[:, None], scatter_idx].set(
        samples, mode='drop', unique_indices=True)
    return new_cn.T.reshape(b, t, c)
